# baseline (speedup 1.0000x reference)
import functools

import jax
import jax.numpy as jnp
from jax import lax
from jax.experimental import pallas as pl
from jax.experimental.pallas import tpu as pltpu
from jax.experimental.pallas import tpu_sc as plsc


_LANES = 16


@functools.lru_cache(maxsize=None)
def _build_embed_kernel(B, L, V, D):
    info = plsc.get_sparse_core_info()
    NC, NS = info.num_cores, info.num_subcores
    NW = NC * NS
    assert L % NW == 0
    PPW = L // NW
    CH = 32
    assert PPW % CH == 0
    NCHUNK = B * (PPW // CH)
    assert NCHUNK % 2 == 0
    HPW = PPW // CH
    assert D % _LANES == 0

    mesh = plsc.VectorSubcoreMesh(core_axis_name="c", subcore_axis_name="s")

    @functools.partial(
        pl.kernel,
        out_type=jax.ShapeDtypeStruct((B * L, D), jnp.float32),
        mesh=mesh,
        scratch_types=[
            pltpu.VMEM((B * PPW,), jnp.int32),
            pltpu.VMEM((PPW, D), jnp.float32),
            [pltpu.VMEM((CH, D), jnp.float32) for _ in range(2)],
            pltpu.SemaphoreType.DMA,
            pltpu.SemaphoreType.DMA,
            [pltpu.SemaphoreType.DMA for _ in range(2)],
            [pltpu.SemaphoreType.DMA for _ in range(2)],
        ],
    )
    def embed(tok_hbm, tab_hbm, pos_hbm, out_hbm,
              idx_v, pos_v, rows_v, isem, psem, gsem, osem):
        wid = lax.axis_index("s") * NC + lax.axis_index("c")
        l0 = wid * PPW

        idx_ds = [
            pltpu.async_copy(tok_hbm.at[bi, pl.ds(l0, PPW)],
                             idx_v.at[pl.ds(bi * PPW, PPW)], isem)
            for bi in range(B)
        ]
        pos_d = pltpu.async_copy(pos_hbm.at[pl.ds(l0, PPW)], pos_v, psem)

        def start_gather(k, s):
            return pltpu.async_copy(
                tab_hbm.at[idx_v.at[pl.ds(k * CH, CH)]], rows_v[s],
                gsem[s])

        for d in idx_ds:
            d.wait()
        in_d = [start_gather(0, 0), None]
        pos_d.wait()

        out_d = [None, None]
        for k in range(NCHUNK):
            s = k & 1
            n = s ^ 1
            if k + 1 < NCHUNK:
                if out_d[n] is not None:
                    out_d[n].wait()
                in_d[n] = start_gather(k + 1, n)
            in_d[s].wait()
            p0 = (k % HPW) * CH

            @pl.loop(0, CH)
            def _row(r):
                for j in range(D // _LANES):
                    sl = pl.ds(j * _LANES, _LANES)
                    rows_v[s][r, sl] = rows_v[s][r, sl] + pos_v[p0 + r, sl]

            out_d[s] = pltpu.async_copy(
                rows_v[s],
                out_hbm.at[pl.ds((k // HPW) * L + l0 + p0, CH)],
                osem[s])

        out_d[0].wait()
        out_d[1].wait()

    return embed


def kernel(tokens, token_table, pos_table):
    B, L = tokens.shape
    V, D = token_table.shape
    embed = _build_embed_kernel(B, L, V, D)
    return embed(tokens, token_table, pos_table).reshape(B, L, D)


# speedup vs baseline: 1.4267x; 1.4267x over previous
import functools

import jax
import jax.numpy as jnp
from jax import lax
from jax.experimental import pallas as pl
from jax.experimental.pallas import tpu as pltpu
from jax.experimental.pallas import tpu_sc as plsc


_LANES = 16


@functools.lru_cache(maxsize=None)
def _build_embed_kernel(B, L, V, D):
    info = plsc.get_sparse_core_info()
    NC, NS = info.num_cores, info.num_subcores
    NW = NC * NS
    assert L % NW == 0
    PPW = L // NW
    CH = PPW // 2
    NCHUNK = 2 * B
    assert D % _LANES == 0

    mesh = plsc.VectorSubcoreMesh(core_axis_name="c", subcore_axis_name="s")

    @functools.partial(
        pl.kernel,
        out_type=jax.ShapeDtypeStruct((B * L, D), jnp.float32),
        mesh=mesh,
        scratch_types=[
            pltpu.VMEM((B * PPW,), jnp.int32),
            [pltpu.VMEM((CH, D), jnp.float32) for _ in range(2)],
            [pltpu.VMEM((CH, D), jnp.float32) for _ in range(2)],
            pltpu.SemaphoreType.DMA,
            [pltpu.SemaphoreType.DMA for _ in range(2)],
            [pltpu.SemaphoreType.DMA for _ in range(2)],
            [pltpu.SemaphoreType.DMA for _ in range(2)],
        ],
    )
    def embed(tok_hbm, tab_hbm, pos_hbm, out_hbm,
              idx_v, pos_v, rows_v, isem, psem, gsem, osem):
        wid = lax.axis_index("s") * NC + lax.axis_index("c")
        l0 = wid * PPW

        idx_ds = [
            pltpu.async_copy(tok_hbm.at[bi, pl.ds(l0, PPW)],
                             idx_v.at[pl.ds(bi * PPW, PPW)], isem)
            for bi in range(B)
        ]
        pos_ds = [
            pltpu.async_copy(pos_hbm.at[pl.ds(l0 + s * CH, CH)], pos_v[s],
                             psem[s])
            for s in range(2)
        ]

        def start_gather(k, s):
            return pltpu.async_copy(
                tab_hbm.at[idx_v.at[pl.ds(k * CH, CH)]], rows_v[s], gsem[s])

        for d in idx_ds:
            d.wait()
        in_d = [start_gather(0, 0), None]
        for d in pos_ds:
            d.wait()

        out_d = [None, None]
        for k in range(NCHUNK):
            s = k & 1
            n = s ^ 1
            if k + 1 < NCHUNK:
                if out_d[n] is not None:
                    out_d[n].wait()
                in_d[n] = start_gather(k + 1, n)
            in_d[s].wait()

            @pl.loop(0, CH)
            def _row(r):
                for j in range(D // _LANES):
                    sl = pl.ds(j * _LANES, _LANES)
                    rows_v[s][r, sl] = rows_v[s][r, sl] + pos_v[s][r, sl]

            out_d[s] = pltpu.async_copy(
                rows_v[s],
                out_hbm.at[pl.ds((k // 2) * L + l0 + s * CH, CH)],
                osem[s])

        out_d[0].wait()
        out_d[1].wait()

    return embed


def kernel(tokens, token_table, pos_table):
    B, L = tokens.shape
    V, D = token_table.shape
    embed = _build_embed_kernel(B, L, V, D)
    return embed(tokens, token_table, pos_table).reshape(B, L, D)


# speedup vs baseline: 1.5325x; 1.0742x over previous
import functools

import jax
import jax.numpy as jnp
from jax import lax
from jax.experimental import pallas as pl
from jax.experimental.pallas import tpu as pltpu
from jax.experimental.pallas import tpu_sc as plsc


_LANES = 16


@functools.lru_cache(maxsize=None)
def _build_embed_kernel(B, L, V, D):
    info = plsc.get_sparse_core_info()
    NC, NS = info.num_cores, info.num_subcores
    NW = NC * NS
    assert L % NW == 0
    PPW = L // NW
    CH = PPW // 2
    NCHUNK = 2 * B
    assert D % _LANES == 0

    mesh = plsc.VectorSubcoreMesh(core_axis_name="c", subcore_axis_name="s")

    @functools.partial(
        pl.kernel,
        out_type=jax.ShapeDtypeStruct((B * L, D), jnp.float32),
        mesh=mesh,
        scratch_types=[
            pltpu.VMEM((B * PPW,), jnp.int32),
            [pltpu.VMEM((CH, D), jnp.float32) for _ in range(2)],
            [pltpu.VMEM((CH, D), jnp.float32) for _ in range(2)],
            pltpu.SemaphoreType.DMA,
            [pltpu.SemaphoreType.DMA for _ in range(2)],
            [pltpu.SemaphoreType.DMA for _ in range(2)],
            [pltpu.SemaphoreType.DMA for _ in range(2)],
        ],
    )
    def embed(tok_hbm, tab_hbm, pos_hbm, out_hbm,
              idx_v, pos_v, rows_v, isem, psem, gsem, osem):
        wid = lax.axis_index("s") * NC + lax.axis_index("c")
        l0 = wid * PPW

        idx_ds = [
            pltpu.async_copy(tok_hbm.at[bi, pl.ds(l0, PPW)],
                             idx_v.at[pl.ds(bi * PPW, PPW)], isem)
            for bi in range(B)
        ]
        pos_ds = [
            pltpu.async_copy(pos_hbm.at[pl.ds(l0 + s * CH, CH)], pos_v[s],
                             psem[s])
            for s in range(2)
        ]

        def start_gather(k, s):
            return pltpu.async_copy(
                tab_hbm.at[idx_v.at[pl.ds(k * CH, CH)]], rows_v[s], gsem[s])

        for d in idx_ds:
            d.wait()
        in_d = [start_gather(0, 0), None]
        for d in pos_ds:
            d.wait()

        def wait_out(s):
            pltpu.make_async_copy(
                rows_v[s], out_hbm.at[pl.ds(0, CH)], osem[s]).wait()

        def wait_gather(s):
            pltpu.make_async_copy(
                tab_hbm.at[idx_v.at[pl.ds(0, CH)]], rows_v[s],
                gsem[s]).wait()

        @pl.loop(0, NCHUNK, step=2)
        def _pair(c):
            for s in range(2):
                k = c + s
                n = s ^ 1

                @pl.when(k + 1 < NCHUNK)
                def _prefetch():
                    @pl.when(k >= 1)
                    def _drain():
                        wait_out(n)
                    start_gather(k + 1, n)

                wait_gather(s)

                @pl.loop(0, CH)
                def _row(r):
                    for j in range(D // _LANES):
                        sl = pl.ds(j * _LANES, _LANES)
                        rows_v[s][r, sl] = rows_v[s][r, sl] + pos_v[s][r, sl]

                pltpu.async_copy(
                    rows_v[s],
                    out_hbm.at[pl.ds((k // 2) * L + l0 + s * CH, CH)],
                    osem[s])

        wait_out(0)
        wait_out(1)

    return embed


def kernel(tokens, token_table, pos_table):
    B, L = tokens.shape
    V, D = token_table.shape
    embed = _build_embed_kernel(B, L, V, D)
    return embed(tokens, token_table, pos_table).reshape(B, L, D)
